# baseline (speedup 1.0000x reference)
import functools

import jax
import jax.numpy as jnp
from jax import lax
from jax.experimental import pallas as pl
from jax.experimental.pallas import tpu as pltpu
from jax.experimental.pallas import tpu_sc as plsc

B = 4
N = 4096
D = 4096
K = N // 2
ROWS = B * K

NC = 2
NS = 16
NW = NC * NS
RPW = ROWS // NW
C = 8
NCHUNK = RPW // C


_Q = N // 128
_KQ = K // 128


def _swap_dist(x, d):
    if d < 128:
        left = jnp.concatenate([x[:, d:], x[:, :d]], axis=1)
        right = jnp.concatenate([x[:, -d:], x[:, :-d]], axis=1)
        lane = lax.broadcasted_iota(jnp.int32, x.shape, 1)
        return jnp.where((lane & d) == 0, left, right)
    dq = d // 128
    left = jnp.concatenate([x[dq:, :], x[:dq, :]], axis=0)
    right = jnp.concatenate([x[-dq:, :], x[:-dq, :]], axis=0)
    sub = lax.broadcasted_iota(jnp.int32, x.shape, 0)
    return jnp.where((sub & dq) == 0, left, right)


def _topk_body(s_ref, vals_ref, idx_ref, gid_ref):
    v = s_ref[...]
    u = lax.bitcast_convert_type(v, jnp.int32)
    key = jnp.where(u < 0, u ^ jnp.int32(0x7FFFFFFF), u)
    sub = lax.broadcasted_iota(jnp.int32, (128, 128), 0)
    lane = lax.broadcasted_iota(jnp.int32, (128, 128), 1)
    elem = (sub % _Q) * 128 + lane
    idx = elem

    for L in range(1, 13):
        kbit = 1 << L
        for d in (1 << p for p in range(L - 1, -1, -1)):
            kp = _swap_dist(key, d)
            ip = _swap_dist(idx, d)
            less = (key > kp) | ((key == kp) & (idx < ip))
            if kbit == N * 2:
                take_hi = (elem & d) != 0
            else:
                take_hi = ((elem & kbit) != 0) ^ ((elem & d) != 0)
            cond = less ^ take_hi
            key = jnp.where(cond, key, kp)
            idx = jnp.where(cond, idx, ip)

    uo = jnp.where(key < 0, key ^ jnp.int32(0x7FFFFFFF), key)
    vo = lax.bitcast_convert_type(uo, jnp.float32)
    for r in range(B):
        vals_ref[r * _KQ:(r + 1) * _KQ, :] = vo[r * _Q:r * _Q + _KQ, :]
        ii = idx[r * _Q:r * _Q + _KQ, :]
        idx_ref[r * _KQ:(r + 1) * _KQ, :] = ii
        gid_ref[r * _KQ:(r + 1) * _KQ, :] = ii + r * N


def _topk(scores):
    vals, idx, gid = pl.pallas_call(
        _topk_body,
        out_shape=[
            jax.ShapeDtypeStruct((B * _KQ, 128), jnp.float32),
            jax.ShapeDtypeStruct((B * _KQ, 128), jnp.int32),
            jax.ShapeDtypeStruct((B * _KQ, 128), jnp.int32),
        ],
    )(scores.reshape(128, 128))
    return (vals.reshape(B, K), idx.reshape(B, K), gid.reshape(B, K))


def _gather_body(gid_hbm, hs_hbm, out_hbm, idx_v, buf0, buf1, buf2,
                 g0, g1, g2, w0, w1, w2):
    wid = lax.axis_index("s") * NC + lax.axis_index("c")
    base = wid * RPW
    pltpu.sync_copy(gid_hbm.at[pl.ds(base, RPW)], idx_v)

    bufs = (buf0, buf1, buf2)
    gsems = (g0, g1, g2)
    wsems = (w0, w1, w2)

    def start_gather(c, s):
        pltpu.async_copy(
            hs_hbm.at[idx_v.at[pl.ds(c * C, C)]], bufs[s], gsems[s])

    def wait_gather(c, s):
        pltpu.make_async_copy(
            hs_hbm.at[idx_v.at[pl.ds(c * C, C)]], bufs[s], gsems[s]).wait()

    def start_write(c, s):
        pltpu.async_copy(
            bufs[s], out_hbm.at[pl.ds(base + c * C, C)], wsems[s])

    def wait_write(c, s):
        pltpu.make_async_copy(
            bufs[s], out_hbm.at[pl.ds(base + c * C, C)], wsems[s]).wait()

    start_gather(0, 0)
    start_gather(1, 1)

    wait_gather(0, 0)
    start_write(0, 0)
    start_gather(2, 2)

    wait_gather(1, 1)
    start_write(1, 1)
    wait_write(0, 0)
    start_gather(3, 0)

    def step(c, s):
        wait_gather(c, s)
        start_write(c, s)
        wait_write(c - 1, (s + 2) % 3)
        start_gather(c + 2, (s + 2) % 3)

    def loop_body(it, _):
        c0 = 2 + it * 3
        step(c0, 2)
        step(c0 + 1, 0)
        step(c0 + 2, 1)
        return _

    lax.fori_loop(0, (NCHUNK - 5) // 3, loop_body, None)

    c = NCHUNK - 3
    wait_gather(c, 2)
    start_write(c, 2)
    wait_write(c - 1, 1)
    start_gather(c + 2, 1)

    wait_gather(NCHUNK - 2, 0)
    start_write(NCHUNK - 2, 0)
    wait_write(NCHUNK - 3, 2)

    wait_gather(NCHUNK - 1, 1)
    start_write(NCHUNK - 1, 1)
    wait_write(NCHUNK - 2, 0)
    wait_write(NCHUNK - 1, 1)


def _gather(hs_flat, gids):
    mesh = plsc.VectorSubcoreMesh(
        core_axis_name="c", subcore_axis_name="s", num_cores=NC,
        num_subcores=NS)
    run = pl.kernel(
        _gather_body,
        out_type=jax.ShapeDtypeStruct((ROWS, D), jnp.float32),
        mesh=mesh,
        scratch_types=[
            pltpu.VMEM((RPW,), jnp.int32),
            pltpu.VMEM((C, D), jnp.float32),
            pltpu.VMEM((C, D), jnp.float32),
            pltpu.VMEM((C, D), jnp.float32),
            pltpu.SemaphoreType.DMA,
            pltpu.SemaphoreType.DMA,
            pltpu.SemaphoreType.DMA,
            pltpu.SemaphoreType.DMA,
            pltpu.SemaphoreType.DMA,
            pltpu.SemaphoreType.DMA,
        ],
    )
    return run(gids, hs_flat)


def kernel(scores, hidden_states):
    vals, idx, gid = _topk(scores)
    gids = gid.reshape(-1)
    selected = _gather(hidden_states.reshape(B * N, D), gids)
    batch_idx = jnp.broadcast_to(
        jnp.arange(B, dtype=jnp.int32)[:, None], (B, K)).reshape(-1)
    return (selected, batch_idx, idx.reshape(-1), vals.reshape(-1))


# speedup vs baseline: 1.0262x; 1.0262x over previous
import functools

import jax
import jax.numpy as jnp
from jax import lax
from jax.experimental import pallas as pl
from jax.experimental.pallas import tpu as pltpu
from jax.experimental.pallas import tpu_sc as plsc

B = 4
N = 4096
D = 4096
K = N // 2
ROWS = B * K

NC = 2
NS = 16
NW = NC * NS
RPW = ROWS // NW
C = 8
NCHUNK = RPW // C


_Q = N // 128
_KQ = K // 128


def _rolls(x, d):
    if d < 128:
        left = jnp.concatenate([x[:, d:], x[:, :d]], axis=1)
        right = jnp.concatenate([x[:, -d:], x[:, :-d]], axis=1)
    else:
        dq = d // 128
        left = jnp.concatenate([x[dq:, :], x[:dq, :]], axis=0)
        right = jnp.concatenate([x[-dq:, :], x[:-dq, :]], axis=0)
    return left, right


def _cmpex(key, idx, d, d_mask, flip_mask):
    kl, kr = _rolls(key, d)
    il, ir = _rolls(idx, d)
    kp = jnp.where(d_mask, kr, kl)
    ip = jnp.where(d_mask, ir, il)
    less = (key > kp) | ((key == kp) & (idx < ip))
    cond = less ^ flip_mask
    return jnp.where(cond, key, kp), jnp.where(cond, idx, ip)


def _topk_body(s_ref, vals_ref, idx_ref, gid_ref):
    sub = lax.broadcasted_iota(jnp.int32, (_Q, 128), 0)
    lane = lax.broadcasted_iota(jnp.int32, (_Q, 128), 1)
    elem = sub * 128 + lane
    u_all = lax.bitcast_convert_type(s_ref[...], jnp.int32)
    key_all = jnp.where(u_all < 0, u_all ^ jnp.int32(0x7FFFFFFF), u_all)
    keys, idxs = [], []
    for r in range(B):
        keys.append(key_all[r * _Q:(r + 1) * _Q, :])
        idxs.append(elem)

    for L in range(1, 12):
        kb_mask = (elem & (1 << L)) != 0
        for d in (1 << p for p in range(L - 1, -1, -1)):
            d_mask = (elem & d) != 0
            flip = kb_mask ^ d_mask
            for r in range(B):
                keys[r], idxs[r] = _cmpex(keys[r], idxs[r], d, d_mask, flip)

    d_mask = (elem & 2048) != 0
    for r in range(B):
        keys[r], idxs[r] = _cmpex(keys[r], idxs[r], 2048, d_mask, d_mask)
    keys = [k[:_KQ, :] for k in keys]
    idxs = [i[:_KQ, :] for i in idxs]

    elem_h = elem[:_KQ, :]
    for d in (1 << p for p in range(10, -1, -1)):
        d_mask = (elem_h & d) != 0
        for r in range(B):
            keys[r], idxs[r] = _cmpex(keys[r], idxs[r], d, d_mask, d_mask)

    ko = jnp.concatenate([k[:_KQ, :] for k in keys], axis=0)
    io = jnp.concatenate([i[:_KQ, :] for i in idxs], axis=0)
    uo = jnp.where(ko < 0, ko ^ jnp.int32(0x7FFFFFFF), ko)
    vals_ref[...] = lax.bitcast_convert_type(uo, jnp.float32)
    idx_ref[...] = io
    sub_o = lax.broadcasted_iota(jnp.int32, (B * _KQ, 128), 0)
    gid_ref[...] = io + (sub_o // _KQ) * N


def _topk(scores):
    vals, idx, gid = pl.pallas_call(
        _topk_body,
        out_shape=[
            jax.ShapeDtypeStruct((B * _KQ, 128), jnp.float32),
            jax.ShapeDtypeStruct((B * _KQ, 128), jnp.int32),
            jax.ShapeDtypeStruct((B * _KQ, 128), jnp.int32),
        ],
    )(scores.reshape(128, 128))
    return (vals.reshape(B, K), idx.reshape(B, K), gid.reshape(B, K))


def _gather_body(gid_hbm, hs_hbm, out_hbm, idx_v, buf0, buf1, buf2,
                 g0, g1, g2, w0, w1, w2):
    wid = lax.axis_index("s") * NC + lax.axis_index("c")
    base = wid * RPW
    pltpu.sync_copy(gid_hbm.at[pl.ds(base, RPW)], idx_v)

    bufs = (buf0, buf1, buf2)
    gsems = (g0, g1, g2)
    wsems = (w0, w1, w2)

    def start_gather(c, s):
        pltpu.async_copy(
            hs_hbm.at[idx_v.at[pl.ds(c * C, C)]], bufs[s], gsems[s])

    def wait_gather(c, s):
        pltpu.make_async_copy(
            hs_hbm.at[idx_v.at[pl.ds(c * C, C)]], bufs[s], gsems[s]).wait()

    def start_write(c, s):
        pltpu.async_copy(
            bufs[s], out_hbm.at[pl.ds(base + c * C, C)], wsems[s])

    def wait_write(c, s):
        pltpu.make_async_copy(
            bufs[s], out_hbm.at[pl.ds(base + c * C, C)], wsems[s]).wait()

    start_gather(0, 0)
    start_gather(1, 1)

    wait_gather(0, 0)
    start_write(0, 0)
    start_gather(2, 2)

    wait_gather(1, 1)
    start_write(1, 1)
    wait_write(0, 0)
    start_gather(3, 0)

    def step(c, s):
        wait_gather(c, s)
        start_write(c, s)
        wait_write(c - 1, (s + 2) % 3)
        start_gather(c + 2, (s + 2) % 3)

    def loop_body(it, _):
        c0 = 2 + it * 3
        step(c0, 2)
        step(c0 + 1, 0)
        step(c0 + 2, 1)
        return _

    lax.fori_loop(0, (NCHUNK - 5) // 3, loop_body, None)

    c = NCHUNK - 3
    wait_gather(c, 2)
    start_write(c, 2)
    wait_write(c - 1, 1)
    start_gather(c + 2, 1)

    wait_gather(NCHUNK - 2, 0)
    start_write(NCHUNK - 2, 0)
    wait_write(NCHUNK - 3, 2)

    wait_gather(NCHUNK - 1, 1)
    start_write(NCHUNK - 1, 1)
    wait_write(NCHUNK - 2, 0)
    wait_write(NCHUNK - 1, 1)


def _gather(hs_flat, gids):
    mesh = plsc.VectorSubcoreMesh(
        core_axis_name="c", subcore_axis_name="s", num_cores=NC,
        num_subcores=NS)
    run = pl.kernel(
        _gather_body,
        out_type=jax.ShapeDtypeStruct((ROWS, D), jnp.float32),
        mesh=mesh,
        scratch_types=[
            pltpu.VMEM((RPW,), jnp.int32),
            pltpu.VMEM((C, D), jnp.float32),
            pltpu.VMEM((C, D), jnp.float32),
            pltpu.VMEM((C, D), jnp.float32),
            pltpu.SemaphoreType.DMA,
            pltpu.SemaphoreType.DMA,
            pltpu.SemaphoreType.DMA,
            pltpu.SemaphoreType.DMA,
            pltpu.SemaphoreType.DMA,
            pltpu.SemaphoreType.DMA,
        ],
    )
    return run(gids, hs_flat)


def kernel(scores, hidden_states):
    vals, idx, gid = _topk(scores)
    gids = gid.reshape(-1)
    selected = _gather(hidden_states.reshape(B * N, D), gids)
    batch_idx = jnp.broadcast_to(
        jnp.arange(B, dtype=jnp.int32)[:, None], (B, K)).reshape(-1)
    return (selected, batch_idx, idx.reshape(-1), vals.reshape(-1))


# speedup vs baseline: 1.0386x; 1.0121x over previous
import functools

import jax
import jax.numpy as jnp
from jax import lax
from jax.experimental import pallas as pl
from jax.experimental.pallas import tpu as pltpu
from jax.experimental.pallas import tpu_sc as plsc

B = 4
N = 4096
D = 4096
K = N // 2
ROWS = B * K

NC = 2
NS = 16
NW = NC * NS
RPW = ROWS // NW
C = 8
NCHUNK = RPW // C


_Q = N // 128
_KQ = K // 128


def _rolls(x, d):
    if d < 128:
        left = jnp.concatenate([x[:, d:], x[:, :d]], axis=1)
        right = jnp.concatenate([x[:, -d:], x[:, :-d]], axis=1)
    else:
        dq = d // 128
        left = jnp.concatenate([x[dq:, :], x[:dq, :]], axis=0)
        right = jnp.concatenate([x[-dq:, :], x[:-dq, :]], axis=0)
    return left, right


def _cmpex(key, idx, d, d_mask, flip_mask):
    kl, kr = _rolls(key, d)
    il, ir = _rolls(idx, d)
    kp = jnp.where(d_mask, kr, kl)
    ip = jnp.where(d_mask, ir, il)
    less = (key > kp) | ((key == kp) & (idx < ip))
    cond = less ^ flip_mask
    return jnp.where(cond, key, kp), jnp.where(cond, idx, ip)


def _topk_body(s_ref, vals_ref, idx_ref, gid_ref):
    sub = lax.broadcasted_iota(jnp.int32, (_Q, 128), 0)
    lane = lax.broadcasted_iota(jnp.int32, (_Q, 128), 1)
    elem = sub * 128 + lane
    u_all = lax.bitcast_convert_type(s_ref[...], jnp.int32)
    key_all = jnp.where(u_all < 0, u_all ^ jnp.int32(0x7FFFFFFF), u_all)
    keys, idxs = [], []
    for r in range(B):
        keys.append(key_all[r * _Q:(r + 1) * _Q, :])
        idxs.append(elem)

    for L in range(1, 12):
        kb_mask = (elem & (1 << L)) != 0
        for d in (1 << p for p in range(L - 1, -1, -1)):
            d_mask = (elem & d) != 0
            flip = kb_mask ^ d_mask
            for r in range(B):
                keys[r], idxs[r] = _cmpex(keys[r], idxs[r], d, d_mask, flip)

    d_mask = (elem & 2048) != 0
    for r in range(B):
        keys[r], idxs[r] = _cmpex(keys[r], idxs[r], 2048, d_mask, d_mask)
    keys = [k[:_KQ, :] for k in keys]
    idxs = [i[:_KQ, :] for i in idxs]

    elem_h = elem[:_KQ, :]
    for d in (1 << p for p in range(10, -1, -1)):
        d_mask = (elem_h & d) != 0
        for r in range(B):
            keys[r], idxs[r] = _cmpex(keys[r], idxs[r], d, d_mask, d_mask)

    ko = jnp.concatenate([k[:_KQ, :] for k in keys], axis=0)
    io = jnp.concatenate([i[:_KQ, :] for i in idxs], axis=0)
    uo = jnp.where(ko < 0, ko ^ jnp.int32(0x7FFFFFFF), ko)
    vals_ref[...] = lax.bitcast_convert_type(uo, jnp.float32)
    idx_ref[...] = io
    sub_o = lax.broadcasted_iota(jnp.int32, (B * _KQ, 128), 0)
    gid_ref[...] = io + (sub_o // _KQ) * N


def _topk(scores):
    vals, idx, gid = pl.pallas_call(
        _topk_body,
        out_shape=[
            jax.ShapeDtypeStruct((B * _KQ, 128), jnp.float32),
            jax.ShapeDtypeStruct((B * _KQ, 128), jnp.int32),
            jax.ShapeDtypeStruct((B * _KQ, 128), jnp.int32),
        ],
    )(scores.reshape(128, 128))
    return (vals.reshape(B, K), idx.reshape(B, K), gid.reshape(B, K))


def _gather_body(gid_hbm, hs_hbm, out_hbm, idx_v, buf0, buf1, g0, g1, w0, w1):
    wid = lax.axis_index("s") * NC + lax.axis_index("c")
    base = wid * RPW
    pltpu.sync_copy(gid_hbm.at[pl.ds(base, RPW)], idx_v)

    bufs = (buf0, buf1)
    gsems = (g0, g1)
    wsems = (w0, w1)

    def start_gather(c, s):
        pltpu.async_copy(
            hs_hbm.at[idx_v.at[pl.ds(c * C, C)]], bufs[s], gsems[s])

    def wait_gather(c, s):
        pltpu.make_async_copy(
            hs_hbm.at[idx_v.at[pl.ds(c * C, C)]], bufs[s], gsems[s]).wait()

    def start_write(c, s):
        pltpu.async_copy(
            bufs[s], out_hbm.at[pl.ds(base + c * C, C)], wsems[s])

    def wait_write(c, s):
        pltpu.make_async_copy(
            bufs[s], out_hbm.at[pl.ds(base + c * C, C)], wsems[s]).wait()

    start_gather(0, 0)
    start_gather(1, 1)

    def loop_body(it, _):
        c0 = it * 2
        for s in (0, 1):
            c = c0 + s
            wait_gather(c, s)
            start_write(c, s)
            wait_write(c, s)
            start_gather(c + 2, s)
        return _

    lax.fori_loop(0, (NCHUNK - 2) // 2, loop_body, None)
    for s in (0, 1):
        c = NCHUNK - 2 + s
        wait_gather(c, s)
        start_write(c, s)
        wait_write(c, s)


def _gather(hs_flat, gids):
    mesh = plsc.VectorSubcoreMesh(
        core_axis_name="c", subcore_axis_name="s", num_cores=NC,
        num_subcores=NS)
    run = pl.kernel(
        _gather_body,
        out_type=jax.ShapeDtypeStruct((ROWS, D), jnp.float32),
        mesh=mesh,
        scratch_types=[
            pltpu.VMEM((RPW,), jnp.int32),
            pltpu.VMEM((C, D), jnp.float32),
            pltpu.VMEM((C, D), jnp.float32),
            pltpu.SemaphoreType.DMA,
            pltpu.SemaphoreType.DMA,
            pltpu.SemaphoreType.DMA,
            pltpu.SemaphoreType.DMA,
        ],
    )
    return run(gids, hs_flat)


def kernel(scores, hidden_states):
    vals, idx, gid = _topk(scores)
    gids = gid.reshape(-1)
    selected = _gather(hidden_states.reshape(B * N, D), gids)
    batch_idx = jnp.broadcast_to(
        jnp.arange(B, dtype=jnp.int32)[:, None], (B, K)).reshape(-1)
    return (selected, batch_idx, idx.reshape(-1), vals.reshape(-1))
